# baseline (speedup 1.0000x reference)
import functools

import jax
import jax.numpy as jnp
from jax import lax
from jax.experimental import pallas as pl
from jax.experimental.pallas import tpu as pltpu
from jax.experimental.pallas import tpu_sc as plsc

_LANES = 16
_GRP = 128
_WPW = 4
_WINL = _WPW * _GRP
_NBUF = 3
_CAP = 32
_LCAP = 1024

_info = plsc.get_sparse_core_info()
_NC, _NS = _info.num_cores, _info.num_subcores
_NW = _NC * _NS


@functools.lru_cache(maxsize=None)
def _make_gather(B: int, D: int, V: int):
    ngrp = (V + _GRP - 1) // _GRP
    gpw = (ngrp + _NW - 1) // _NW
    nwin = (gpw + _WPW - 1) // _WPW
    max_lane0 = (ngrp - _WPW) * _GRP
    nscan = B // _LANES

    def body(labels_hbm, table_t_hbm, out_hbm,
             lbl_all, mv_v, mp_v, bkt_v, bkt_p, blks, stage, pos_v,
             cnt_s, sem, sem2):
        wid = lax.axis_index("s") * _NC + lax.axis_index("c")
        g0 = wid * gpw
        g1 = g0 + gpw
        iota = lax.iota(jnp.int32, _LANES)
        lane0_mask = iota == 0

        def lane0(w):
            return jnp.minimum((g0 + w * _WPW) * _GRP, max_lane0)

        def fire(w):
            pltpu.async_copy(
                table_t_hbm.at[:, pl.ds(lane0(w), _WINL)],
                blks.at[w % _NBUF],
                sem,
            )

        def drain(w):
            pltpu.make_async_copy(
                table_t_hbm.at[:, pl.ds(0, _WINL)],
                blks.at[w % _NBUF],
                sem,
            ).wait()

        for w in range(_NBUF):
            fire(jnp.int32(w))

        pltpu.sync_copy(labels_hbm, lbl_all)

        def prefill(q, _):
            mv_v[pl.ds(q * _LANES, _LANES)] = jnp.full(
                (_LANES,), jnp.int32(1 << 29), jnp.int32)
            return 0
        lax.fori_loop(0, (_LCAP + _LANES) // _LANES, prefill, 0)

        def zero(w, _):
            cnt_s[w] = jnp.int32(0)
            return 0
        lax.fori_loop(0, nwin, zero, 0)

        def scan(q, cnt):
            vec = lbl_all[pl.ds(q * _LANES, _LANES)]
            grp = lax.shift_right_logical(vec, 7)
            m = (grp >= g0) & (grp < g1)
            plsc.store_compressed(mv_v.at[pl.ds(cnt, _LANES)], vec, mask=m)
            plsc.store_compressed(mp_v.at[pl.ds(cnt, _LANES)],
                                  iota + q * _LANES, mask=m)
            c = plsc.all_reduce_population_count(m)[0]
            return jnp.minimum(cnt + c, _LCAP)

        cnt = lax.fori_loop(0, nscan, scan, jnp.int32(0))
        mv_v[pl.ds(cnt, _LANES)] = jnp.full((_LANES,), jnp.int32(1 << 29),
                                            jnp.int32)

        def bucket(q, _):
            mvec = mv_v[pl.ds(q * _LANES, _LANES)]
            pvec = mp_v[pl.ds(q * _LANES, _LANES)]
            for t in range(_LANES):
                v = mvec[t]
                p = pvec[t]
                grp = lax.shift_right_logical(v, 7)

                @pl.when((grp >= g0) & (grp < g1))
                def _append():
                    wloc = lax.shift_right_logical(grp - g0, 2)
                    c = cnt_s[wloc]
                    slot = wloc * _CAP + jnp.minimum(c, _CAP - 1)
                    plsc.store_scatter(
                        bkt_v, [jnp.full((_LANES,), slot, jnp.int32)],
                        jnp.full((_LANES,), v, jnp.int32), mask=lane0_mask)
                    plsc.store_scatter(
                        bkt_p, [jnp.full((_LANES,), slot, jnp.int32)],
                        jnp.full((_LANES,), p, jnp.int32), mask=lane0_mask)
                    cnt_s[wloc] = c + 1
            return 0

        lax.fori_loop(0, lax.shift_right_logical(cnt + _LANES - 1, 4),
                      bucket, 0)

        def window(w, _):
            drain(w)
            cw = cnt_s[w]
            bv0 = bkt_v[pl.ds(w * _CAP, _LANES)]
            bv1 = bkt_v[pl.ds(w * _CAP + _LANES, _LANES)]
            bp0 = bkt_p[pl.ds(w * _CAP, _LANES)]
            bp1 = bkt_p[pl.ds(w * _CAP + _LANES, _LANES)]
            l0 = lane0(w)
            for t in range(_CAP):
                v = (bv0 if t < _LANES else bv1)[t % _LANES]
                p = (bp0 if t < _LANES else bp1)[t % _LANES]
                active = t < cw
                pw = jnp.where(active, p, jnp.int32(-1))
                plsc.store_scatter(
                    pos_v, [jnp.full((_LANES,), t, jnp.int32)],
                    jnp.full((_LANES,), pw, jnp.int32), mask=lane0_mask)

                @pl.when(active)
                def _extract():
                    lane = jnp.full((_LANES,), v - l0, jnp.int32)
                    t_splat = jnp.full((_LANES,), t, jnp.int32)
                    for k in range(D // _LANES):
                        r_vec = iota + (k * _LANES)
                        vals = plsc.load_gather(blks.at[w % _NBUF],
                                                [r_vec, lane])
                        plsc.store_scatter(stage, [t_splat, r_vec], vals)

            @pl.when(w + _NBUF < nwin)
            def _prefetch():
                fire(w + _NBUF)

            pltpu.async_copy(
                stage,
                out_hbm.at[plsc.Indices(pos_v, ignored_value=-1)],
                sem2,
            ).wait()
            return 0

        lax.fori_loop(0, nwin, window, 0)

    return pl.kernel(
        body,
        mesh=plsc.VectorSubcoreMesh(core_axis_name="c", subcore_axis_name="s"),
        compiler_params=pltpu.CompilerParams(needs_layout_passes=False),
        out_type=jax.ShapeDtypeStruct((B, _GRP), jnp.float32),
        scratch_types=[
            pltpu.VMEM((B,), jnp.int32),
            pltpu.VMEM((_LCAP + _LANES,), jnp.int32),
            pltpu.VMEM((_LCAP + _LANES,), jnp.int32),
            pltpu.VMEM((nwin * _CAP,), jnp.int32),
            pltpu.VMEM((nwin * _CAP,), jnp.int32),
            pltpu.VMEM((_NBUF, D, _WINL), jnp.float32),
            pltpu.VMEM((_CAP, _GRP), jnp.float32),
            pltpu.VMEM((_CAP,), jnp.int32),
            pltpu.SMEM((64,), jnp.int32),
            pltpu.SemaphoreType.DMA,
            pltpu.SemaphoreType.DMA,
        ],
    )


def kernel(labels, embedding_table):
    B = labels.shape[0]
    V, D = embedding_table.shape
    labels_i = labels.astype(jnp.int32)
    out_wide = _make_gather(B, D, V)(labels_i, embedding_table.T)
    return out_wide[:, :D]


# speedup vs baseline: 1.0372x; 1.0372x over previous
import functools

import jax
import jax.numpy as jnp
from jax import lax
from jax.experimental import pallas as pl
from jax.experimental.pallas import tpu as pltpu
from jax.experimental.pallas import tpu_sc as plsc

_LANES = 16
_GRP = 128
_WPW = 4
_WINL = _WPW * _GRP
_NBUF = 3
_CAP = 32
_LCAP = 1024

_info = plsc.get_sparse_core_info()
_NC, _NS = _info.num_cores, _info.num_subcores
_NW = _NC * _NS


@functools.lru_cache(maxsize=None)
def _make_gather(B: int, D: int, V: int):
    ngrp = (V + _GRP - 1) // _GRP
    gpw = (ngrp + _NW - 1) // _NW
    nwin = (gpw + _WPW - 1) // _WPW
    max_lane0 = (ngrp - _WPW) * _GRP
    nscan = B // _LANES

    def body(labels_hbm, table_t_hbm, out_hbm,
             lbl_all, mv_v, mp_v, bkt_v, bkt_p, blks, stage, pos_v,
             cnt_s, sem, sem2):
        wid = lax.axis_index("s") * _NC + lax.axis_index("c")
        g0 = wid * gpw
        g1 = g0 + gpw
        iota = lax.iota(jnp.int32, _LANES)
        lane0_mask = iota == 0

        def lane0(w):
            return jnp.minimum((g0 + w * _WPW) * _GRP, max_lane0)

        def fire(w):
            pltpu.async_copy(
                table_t_hbm.at[:, pl.ds(lane0(w), _WINL)],
                blks.at[w % _NBUF],
                sem,
            )

        def drain(w):
            pltpu.make_async_copy(
                table_t_hbm.at[:, pl.ds(0, _WINL)],
                blks.at[w % _NBUF],
                sem,
            ).wait()

        for w in range(_NBUF):
            fire(jnp.int32(w))

        pltpu.sync_copy(labels_hbm, lbl_all)

        def prefill(q, _):
            mv_v[pl.ds(q * _LANES, _LANES)] = jnp.full(
                (_LANES,), jnp.int32(1 << 29), jnp.int32)
            return 0
        lax.fori_loop(0, (_LCAP + _LANES) // _LANES, prefill, 0)

        def zero(w, _):
            cnt_s[w] = jnp.int32(0)
            return 0
        lax.fori_loop(0, nwin, zero, 0)

        def scan(q, cnt):
            vec = lbl_all[pl.ds(q * _LANES, _LANES)]
            grp = lax.shift_right_logical(vec, 7)
            m = (grp >= g0) & (grp < g1)
            plsc.store_compressed(mv_v.at[pl.ds(cnt, _LANES)], vec, mask=m)
            plsc.store_compressed(mp_v.at[pl.ds(cnt, _LANES)],
                                  iota + q * _LANES, mask=m)
            c = plsc.all_reduce_population_count(m)[0]
            return jnp.minimum(cnt + c, _LCAP)

        cnt = lax.fori_loop(0, nscan, scan, jnp.int32(0))
        mv_v[pl.ds(cnt, _LANES)] = jnp.full((_LANES,), jnp.int32(1 << 29),
                                            jnp.int32)

        def bucket(q, _):
            mvec = mv_v[pl.ds(q * _LANES, _LANES)]
            pvec = mp_v[pl.ds(q * _LANES, _LANES)]
            for t in range(_LANES):
                v = mvec[t]
                p = pvec[t]
                grp = lax.shift_right_logical(v, 7)

                @pl.when((grp >= g0) & (grp < g1))
                def _append():
                    wloc = lax.shift_right_logical(grp - g0, 2)
                    c = cnt_s[wloc]
                    slot = wloc * _CAP + jnp.minimum(c, _CAP - 1)
                    plsc.store_scatter(
                        bkt_v, [jnp.full((_LANES,), slot, jnp.int32)],
                        jnp.full((_LANES,), v, jnp.int32), mask=lane0_mask)
                    plsc.store_scatter(
                        bkt_p, [jnp.full((_LANES,), slot, jnp.int32)],
                        jnp.full((_LANES,), p, jnp.int32), mask=lane0_mask)
                    cnt_s[wloc] = c + 1
            return 0

        lax.fori_loop(0, lax.shift_right_logical(cnt + _LANES - 1, 4),
                      bucket, 0)

        def window(w, _):
            drain(w)
            cw = cnt_s[w]
            bv0 = bkt_v[pl.ds(w * _CAP, _LANES)]
            bv1 = bkt_v[pl.ds(w * _CAP + _LANES, _LANES)]
            bp0 = bkt_p[pl.ds(w * _CAP, _LANES)]
            bp1 = bkt_p[pl.ds(w * _CAP + _LANES, _LANES)]
            l0 = lane0(w)
            neg1 = jnp.full((_LANES,), jnp.int32(-1), jnp.int32)
            for t0 in range(0, _CAP, _LANES):
                pos_v[pl.ds(t0, _LANES)] = neg1
            for t in range(_CAP):
                active = t < cw

                @pl.when(active)
                def _extract():
                    v = (bv0 if t < _LANES else bv1)[t % _LANES]
                    p = (bp0 if t < _LANES else bp1)[t % _LANES]
                    t_splat = jnp.full((_LANES,), t, jnp.int32)
                    plsc.store_scatter(
                        pos_v, [t_splat],
                        jnp.full((_LANES,), p, jnp.int32), mask=lane0_mask)
                    lane = jnp.full((_LANES,), v - l0, jnp.int32)
                    for k in range(D // _LANES):
                        r_vec = iota + (k * _LANES)
                        vals = plsc.load_gather(blks.at[w % _NBUF],
                                                [r_vec, lane])
                        plsc.store_scatter(stage, [t_splat, r_vec], vals)

            @pl.when(w + _NBUF < nwin)
            def _prefetch():
                fire(w + _NBUF)

            pltpu.async_copy(
                stage,
                out_hbm.at[plsc.Indices(pos_v, ignored_value=-1)],
                sem2,
            ).wait()
            return 0

        lax.fori_loop(0, nwin, window, 0)

    return pl.kernel(
        body,
        mesh=plsc.VectorSubcoreMesh(core_axis_name="c", subcore_axis_name="s"),
        compiler_params=pltpu.CompilerParams(needs_layout_passes=False),
        out_type=jax.ShapeDtypeStruct((B, _GRP), jnp.float32),
        scratch_types=[
            pltpu.VMEM((B,), jnp.int32),
            pltpu.VMEM((_LCAP + _LANES,), jnp.int32),
            pltpu.VMEM((_LCAP + _LANES,), jnp.int32),
            pltpu.VMEM((nwin * _CAP,), jnp.int32),
            pltpu.VMEM((nwin * _CAP,), jnp.int32),
            pltpu.VMEM((_NBUF, D, _WINL), jnp.float32),
            pltpu.VMEM((_CAP, _GRP), jnp.float32),
            pltpu.VMEM((_CAP,), jnp.int32),
            pltpu.SMEM((64,), jnp.int32),
            pltpu.SemaphoreType.DMA,
            pltpu.SemaphoreType.DMA,
        ],
    )


def kernel(labels, embedding_table):
    B = labels.shape[0]
    V, D = embedding_table.shape
    labels_i = labels.astype(jnp.int32)
    out_wide = _make_gather(B, D, V)(labels_i, embedding_table.T)
    return out_wide[:, :D]
